# Initial kernel scaffold; baseline (speedup 1.0000x reference)
import jax
import jax.numpy as jnp
from jax.experimental import pallas as pl


def kernel(edge_index, edge_weight, emb, Wl0, Wr0, att0, b0, Wl1, Wr1, att1, b1, Wl2, Wr2, att2, b2, Wl3, Wr3, att3, b3, Wout, bout):
    raise NotImplementedError("write your pallas kernel here")


# speedup vs baseline: 7.3979x; 7.3979x over previous
import jax
import jax.numpy as jnp
from jax.experimental import pallas as pl

N_NODES = 50000
N_EDGES = 800000
HID = 64
HEADS = 2

_NODE_BLK = 2000
_EDGE_BLK = 8000


def _proj_kernel(x_ref, w_ref, o_ref):
    o_ref[...] = x_ref[...] @ w_ref[...]


def _proj(x, w):
    n, k = x.shape[0], w.shape[1]
    return pl.pallas_call(
        _proj_kernel,
        grid=(n // _NODE_BLK,),
        in_specs=[
            pl.BlockSpec((_NODE_BLK, HID), lambda i: (i, 0)),
            pl.BlockSpec((HID, k), lambda i: (0, 0)),
        ],
        out_specs=pl.BlockSpec((_NODE_BLK, k), lambda i: (i, 0)),
        out_shape=jax.ShapeDtypeStruct((n, k), x.dtype),
    )(x, w)


def _alpha_kernel(xls_ref, xrd_ref, att_ref, o_ref):
    m = xls_ref[...] + xrd_ref[...]
    m = jnp.where(m > 0, m, 0.2 * m)
    ma = m * att_ref[...]
    a0 = jnp.sum(ma[:, :HID], axis=1, keepdims=True)
    a1 = jnp.sum(ma[:, HID:], axis=1, keepdims=True)
    o_ref[...] = jnp.concatenate([a0, a1], axis=1)


def _edge_alpha(xls, xrd, att_flat):
    e = xls.shape[0]
    return pl.pallas_call(
        _alpha_kernel,
        grid=(e // _EDGE_BLK,),
        in_specs=[
            pl.BlockSpec((_EDGE_BLK, HEADS * HID), lambda i: (i, 0)),
            pl.BlockSpec((_EDGE_BLK, HEADS * HID), lambda i: (i, 0)),
            pl.BlockSpec((1, HEADS * HID), lambda i: (0, 0)),
        ],
        out_specs=pl.BlockSpec((_EDGE_BLK, HEADS), lambda i: (i, 0)),
        out_shape=jax.ShapeDtypeStruct((e, HEADS), xls.dtype),
    )(xls, xrd, att_flat)


def _msg_kernel(xls_ref, a_ref, o_ref):
    xls = xls_ref[...]
    a = a_ref[...]
    o_ref[...] = jnp.concatenate(
        [xls[:, :HID] * a[:, 0:1], xls[:, HID:] * a[:, 1:2]], axis=1
    )


def _edge_msg(xls, a):
    e = xls.shape[0]
    return pl.pallas_call(
        _msg_kernel,
        grid=(e // _EDGE_BLK,),
        in_specs=[
            pl.BlockSpec((_EDGE_BLK, HEADS * HID), lambda i: (i, 0)),
            pl.BlockSpec((_EDGE_BLK, HEADS), lambda i: (i, 0)),
        ],
        out_specs=pl.BlockSpec((_EDGE_BLK, HEADS * HID), lambda i: (i, 0)),
        out_shape=jax.ShapeDtypeStruct((e, HEADS * HID), xls.dtype),
    )(xls, a)


def _out_kernel(x_ref, w_ref, b_ref, o_ref):
    y = x_ref[...] @ w_ref[...] + b_ref[...]
    o_ref[...] = jnp.where(y > 0, y, 0.01 * y)


def _out_layer(x, w, b):
    n = x.shape[0]
    return pl.pallas_call(
        _out_kernel,
        grid=(n // _NODE_BLK,),
        in_specs=[
            pl.BlockSpec((_NODE_BLK, HID), lambda i: (i, 0)),
            pl.BlockSpec((HID, HID), lambda i: (0, 0)),
            pl.BlockSpec((1, HID), lambda i: (0, 0)),
        ],
        out_specs=pl.BlockSpec((_NODE_BLK, HID), lambda i: (i, 0)),
        out_shape=jax.ShapeDtypeStruct((n, HID), x.dtype),
    )(x, w, b)


def _gatv2_layer(x, src, dst, Wl, Wr, att, b):
    n = x.shape[0]
    xl = _proj(x, Wl)
    xr = _proj(x, Wr)
    xls = jnp.take(xl, src, axis=0)
    xrd = jnp.take(xr, dst, axis=0)
    att_flat = att.reshape(1, HEADS * HID)
    alpha = _edge_alpha(xls, xrd, att_flat)
    amax = jax.ops.segment_max(alpha, dst, num_segments=n)
    amax = jnp.where(jnp.isfinite(amax), amax, 0.0)
    ex = jnp.exp(alpha - amax[dst])
    denom = jax.ops.segment_sum(ex, dst, num_segments=n)
    a = ex / (denom[dst] + 1e-16)
    msg = _edge_msg(xls, a)
    agg = jax.ops.segment_sum(msg, dst, num_segments=n)
    out = 0.5 * (agg[:, :HID] + agg[:, HID:]) + b
    return jnp.where(out > 0, out, 0.01 * out)


def kernel(edge_index, edge_weight, emb, Wl0, Wr0, att0, b0, Wl1, Wr1, att1, b1, Wl2, Wr2, att2, b2, Wl3, Wr3, att3, b3, Wout, bout):
    src = edge_index[0]
    dst = edge_index[1]
    x = emb
    params = [(Wl0, Wr0, att0, b0), (Wl1, Wr1, att1, b1),
              (Wl2, Wr2, att2, b2), (Wl3, Wr3, att3, b3)]
    for (Wl, Wr, att, b) in params:
        x = _gatv2_layer(x, src, dst, Wl, Wr, att, b)
    return _out_layer(x, Wout, bout.reshape(1, HID))
